# baseline (speedup 1.0000x reference)
import functools

import numpy as np
import jax
import jax.numpy as jnp
from jax import lax
from jax.experimental import pallas as pl
from jax.experimental.pallas import tpu as pltpu
from jax.experimental.pallas import tpu_sc as plsc

_B, _N, _S, _K = 4, 8192, 32, 12
_DS, _DT = 256, 1024
_G = _B * _S
_ROWS = _G * _K
_NW = 32
_RPW = _ROWS // _NW
_GPB = 16
_SQRT_C = float(np.sqrt(1.0 + 1e-5))


def _fps_knn_body(xt1_ref, xt2_ref, xs_ref, it1_ref, it2_ref, is_ref):
    x0 = xt1_ref[:, 0, :]
    x1 = xt1_ref[:, 1, :]
    x2 = xt1_ref[:, 2, :]
    iota = lax.broadcasted_iota(jnp.int32, (_B, _N), 1)
    dist = jnp.full((_B, _N), 1e10, dtype=jnp.float32)
    far = jnp.zeros((_B, 1), dtype=jnp.int32)
    qxs, qys, qzs = [], [], []
    for _ in range(_S):
        msk = iota == far
        c = jnp.sum(jnp.where(msk[:, None, :], xt1_ref[...], 0.0), axis=2)
        cx = c[:, 0:1]
        cy = c[:, 1:2]
        cz = c[:, 2:3]
        qxs.append(cx)
        qys.append(cy)
        qzs.append(cz)
        d = (x0 - cx) ** 2 + (x1 - cy) ** 2 + (x2 - cz) ** 2
        dist = jnp.minimum(dist, d)
        far = jnp.argmax(dist, axis=1).astype(jnp.int32)[:, None]
    qx = jnp.concatenate(qxs, axis=1)
    qy = jnp.concatenate(qys, axis=1)
    qz = jnp.concatenate(qzs, axis=1)
    sq = qx * qx + qy * qy + qz * qz
    qxf = qx.astype(jnp.bfloat16).astype(jnp.float32)
    qyf = qy.astype(jnp.bfloat16).astype(jnp.float32)
    qzf = qz.astype(jnp.bfloat16).astype(jnp.float32)
    iota3 = lax.broadcasted_iota(jnp.int32, (_B, _S, _N), 2)
    for db_ref, out_ref in ((xt1_ref, it1_ref), (xt2_ref, it2_ref),
                            (xs_ref, is_ref)):
        y0 = db_ref[:, 0, :]
        y1 = db_ref[:, 1, :]
        y2 = db_ref[:, 2, :]
        sx = y0 * y0 + y1 * y1 + y2 * y2
        y0f = y0.astype(jnp.bfloat16).astype(jnp.float32)
        y1f = y1.astype(jnp.bfloat16).astype(jnp.float32)
        y2f = y2.astype(jnp.bfloat16).astype(jnp.float32)
        p0 = qxf[:, :, None] * y0f[:, None, :]
        p1 = qyf[:, :, None] * y1f[:, None, :]
        p2 = qzf[:, :, None] * y2f[:, None, :]
        dot = (p0 + p1) + p2
        d = (sq[:, :, None] + sx[:, None, :]) - 2.0 * dot
        cols = []
        m = None
        for k in range(_K):
            dm = d if m is None else jnp.where(m, jnp.float32(jnp.inf), d)
            ik = jnp.argmin(dm, axis=2).astype(jnp.int32)[:, :, None]
            cols.append(ik)
            if k < _K - 1:
                sel = iota3 == ik
                m = sel if m is None else m | sel
        out_ref[...] = jnp.concatenate(cols, axis=2)


def _fps_knn(xt1, xt2, xs):
    out = jax.ShapeDtypeStruct((_B, _S, _K), jnp.int32)
    return pl.pallas_call(
        _fps_knn_body,
        out_shape=(out, out, out),
    )(xt1, xt2, xs)


def _sc_gather_body(t1_ref, t2_ref, ts_ref, idx_ref,
                    o1_ref, o2_ref, os_ref,
                    idx_v, r1_v, r2_v, rs_v,
                    sem1, sem2, sem3, semw1, semw2, semw3):
    wid = lax.axis_index("s") * 2 + lax.axis_index("c")
    base = wid * _RPW
    pltpu.sync_copy(idx_ref.at[pl.ds(wid * 3 * _RPW, 3 * _RPW)], idx_v)
    c1 = pltpu.async_copy(t1_ref.at[idx_v.at[pl.ds(0, _RPW)]], r1_v, sem1)
    c2 = pltpu.async_copy(t2_ref.at[idx_v.at[pl.ds(_RPW, _RPW)]], r2_v, sem2)
    c3 = pltpu.async_copy(ts_ref.at[idx_v.at[pl.ds(2 * _RPW, _RPW)]],
                          rs_v, sem3)
    c1.wait()
    w1 = pltpu.async_copy(r1_v, o1_ref.at[pl.ds(base, _RPW)], semw1)
    c2.wait()
    w2 = pltpu.async_copy(r2_v, o2_ref.at[pl.ds(base, _RPW)], semw2)
    c3.wait()
    w3 = pltpu.async_copy(rs_v, os_ref.at[pl.ds(base, _RPW)], semw3)
    w1.wait()
    w2.wait()
    w3.wait()


@functools.lru_cache(maxsize=1)
def _sc_gather_kernel():
    return pl.kernel(
        _sc_gather_body,
        out_type=(
            jax.ShapeDtypeStruct((_ROWS, _DT), jnp.float32),
            jax.ShapeDtypeStruct((_ROWS, _DT), jnp.float32),
            jax.ShapeDtypeStruct((_ROWS, _DS), jnp.float32),
        ),
        mesh=plsc.VectorSubcoreMesh(core_axis_name="c", subcore_axis_name="s"),
        scratch_types=[
            pltpu.VMEM((3 * _RPW,), jnp.int32),
            pltpu.VMEM((_RPW, _DT), jnp.float32),
            pltpu.VMEM((_RPW, _DT), jnp.float32),
            pltpu.VMEM((_RPW, _DS), jnp.float32),
            pltpu.SemaphoreType.DMA,
            pltpu.SemaphoreType.DMA,
            pltpu.SemaphoreType.DMA,
            pltpu.SemaphoreType.DMA,
            pltpu.SemaphoreType.DMA,
            pltpu.SemaphoreType.DMA,
        ],
    )


def _sc_gather(t1, t2, ts, idx):
    return _sc_gather_kernel()(t1, t2, ts, idx)


def _segmax(y):
    parts = [jnp.max(y[g * _K:(g + 1) * _K, :], axis=0, keepdims=True)
             for g in range(_GPB)]
    return jnp.concatenate(parts, axis=0)


def _bn_relu(y, b, g, be):
    y = y + b
    y = y / _SQRT_C * g + be
    return jnp.maximum(y, 0.0)


def _dense_body(g1_ref, g2_ref, gs_ref, wt1_ref, wt2_ref, wts_ref,
                b1_ref, g1s_ref, be1_ref, b2_ref, g2s_ref, be2_ref,
                bs_ref, gss_ref, bes_ref,
                wf1_ref, wf2_ref, wc1_ref, bc1_ref, gc1_ref, bec1_ref,
                wc2_ref, bc2_ref, gc2_ref, bec2_ref, wc3_ref, bc3_ref,
                os_ref, fused_ref, cls_ref, acc1_ref, acc2_ref):
    i = pl.program_id(0)
    y1 = jnp.dot(g1_ref[...].astype(jnp.bfloat16),
                 wt1_ref[...].astype(jnp.bfloat16),
                 preferred_element_type=jnp.float32)
    s1 = _segmax(_bn_relu(y1, b1_ref[...], g1s_ref[...], be1_ref[...]))
    acc1_ref[pl.ds(i * _GPB, _GPB), :] = s1
    y2 = jnp.dot(g2_ref[...].astype(jnp.bfloat16),
                 wt2_ref[...].astype(jnp.bfloat16),
                 preferred_element_type=jnp.float32)
    s2 = _segmax(_bn_relu(y2, b2_ref[...], g2s_ref[...], be2_ref[...]))
    acc2_ref[pl.ds(i * _GPB, _GPB), :] = s2
    ys = jnp.dot(gs_ref[...].astype(jnp.bfloat16),
                 wts_ref[...].astype(jnp.bfloat16),
                 preferred_element_type=jnp.float32)
    os_ref[...] = _segmax(_bn_relu(ys, bs_ref[...], gss_ref[...],
                                   bes_ref[...]))

    @pl.when(i == _G // _GPB - 1)
    def _():
        t1 = acc1_ref[...].reshape(_B, _S, _DT)
        t2 = acc2_ref[...].reshape(_B, _S, _DT)
        m1 = jnp.mean(t1, axis=1)
        m2 = jnp.mean(t2, axis=1)
        pm = jnp.concatenate([m1, m2], axis=1)
        y = jnp.dot(pm, wf1_ref[...], preferred_element_type=jnp.float32)
        y = jnp.maximum(y, 0.0)
        y = jnp.dot(y, wf2_ref[...], preferred_element_type=jnp.float32)
        mx = jnp.max(y, axis=1, keepdims=True)
        e = jnp.exp(y - mx)
        sm = e / jnp.sum(e, axis=1, keepdims=True)
        w0 = sm[:, 0:1][:, :, None]
        w1 = sm[:, 1:2][:, :, None]
        fused = w0 * t1 + w1 * t2
        fused_ref[...] = fused
        xm = jnp.max(fused, axis=1)
        h = jnp.dot(xm, wc1_ref[...], preferred_element_type=jnp.float32)
        h = h + bc1_ref[...]
        h = jnp.maximum(h / _SQRT_C * gc1_ref[...] + bec1_ref[...], 0.0)
        h = jnp.dot(h, wc2_ref[...], preferred_element_type=jnp.float32)
        h = h + bc2_ref[...]
        h = jnp.maximum(h / _SQRT_C * gc2_ref[...] + bec2_ref[...], 0.0)
        c = jnp.dot(h, wc3_ref[...], preferred_element_type=jnp.float32)
        cls_ref[...] = c + bc3_ref[...]


def _dense(g1, g2, gs, wt1, wt2, wts, bn1, bn2, bns,
           wf1t, wf2t, wc1t, bc1, gc1, bec1, wc2t, bc2, gc2, bec2,
           wc3t, bc3):
    row = pl.BlockSpec((1, _DT), lambda i: (0, 0))
    return pl.pallas_call(
        _dense_body,
        grid=(_G // _GPB,),
        in_specs=[
            pl.BlockSpec((_GPB * _K, _DT), lambda i: (i, 0)),
            pl.BlockSpec((_GPB * _K, _DT), lambda i: (i, 0)),
            pl.BlockSpec((_GPB * _K, _DS), lambda i: (i, 0)),
            pl.BlockSpec((_DT, _DT), lambda i: (0, 0)),
            pl.BlockSpec((_DT, _DT), lambda i: (0, 0)),
            pl.BlockSpec((_DS, _DT), lambda i: (0, 0)),
            row, row, row, row, row, row, row, row, row,
            pl.BlockSpec((2 * _DT, 128), lambda i: (0, 0)),
            pl.BlockSpec((128, 2), lambda i: (0, 0)),
            pl.BlockSpec((_DT, 512), lambda i: (0, 0)),
            pl.BlockSpec((1, 512), lambda i: (0, 0)),
            pl.BlockSpec((1, 512), lambda i: (0, 0)),
            pl.BlockSpec((1, 512), lambda i: (0, 0)),
            pl.BlockSpec((512, 256), lambda i: (0, 0)),
            pl.BlockSpec((1, 256), lambda i: (0, 0)),
            pl.BlockSpec((1, 256), lambda i: (0, 0)),
            pl.BlockSpec((1, 256), lambda i: (0, 0)),
            pl.BlockSpec((256, 15), lambda i: (0, 0)),
            pl.BlockSpec((1, 15), lambda i: (0, 0)),
        ],
        out_specs=(
            pl.BlockSpec((_GPB, _DT), lambda i: (i, 0)),
            pl.BlockSpec((_B, _S, _DT), lambda i: (0, 0, 0)),
            pl.BlockSpec((_B, 15), lambda i: (0, 0)),
        ),
        out_shape=(
            jax.ShapeDtypeStruct((_G, _DT), jnp.float32),
            jax.ShapeDtypeStruct((_B, _S, _DT), jnp.float32),
            jax.ShapeDtypeStruct((_B, 15), jnp.float32),
        ),
        scratch_shapes=[
            pltpu.VMEM((_G, _DT), jnp.float32),
            pltpu.VMEM((_G, _DT), jnp.float32),
        ],
    )(g1, g2, gs, wt1, wt2, wts,
      bn1[0].reshape(1, -1), bn1[1].reshape(1, -1), bn1[2].reshape(1, -1),
      bn2[0].reshape(1, -1), bn2[1].reshape(1, -1), bn2[2].reshape(1, -1),
      bns[0].reshape(1, -1), bns[1].reshape(1, -1), bns[2].reshape(1, -1),
      wf1t, wf2t, wc1t, bc1.reshape(1, -1), gc1.reshape(1, -1),
      bec1.reshape(1, -1), wc2t, bc2.reshape(1, -1), gc2.reshape(1, -1),
      bec2.reshape(1, -1), wc3t, bc3.reshape(1, -1))


def kernel(feature_s, xyz_s, feature_t1, xyz_t1, feature_t2, xyz_t2,
           W_s, b_s, g_s, be_s, W_t1, b_t1, g_t1, be_t1,
           W_t2, b_t2, g_t2, be_t2, Wf1, Wf2, Wc1, bc1, gc1, bec1,
           Wc2, bc2, gc2, bec2, Wc3, bc3):
    xt1 = jnp.transpose(xyz_t1, (0, 2, 1))
    xt2 = jnp.transpose(xyz_t2, (0, 2, 1))
    xs = jnp.transpose(xyz_s, (0, 2, 1))
    idx_t1, idx_t2, idx_s = _fps_knn(xt1, xt2, xs)
    off = (jnp.arange(_B, dtype=jnp.int32) * _N)[:, None, None]
    f1 = (idx_t1 + off).reshape(_NW, _RPW)
    f2 = (idx_t2 + off).reshape(_NW, _RPW)
    fs = (idx_s + off).reshape(_NW, _RPW)
    idx = jnp.stack([f1, f2, fs], axis=1).reshape(-1)
    g1, g2, gs = _sc_gather(
        feature_t1.reshape(_B * _N, _DT),
        feature_t2.reshape(_B * _N, _DT),
        feature_s.reshape(_B * _N, _DS),
        idx)
    out_s, fused, cls = _dense(
        g1, g2, gs, W_t1.T, W_t2.T, W_s.T,
        (b_t1, g_t1, be_t1), (b_t2, g_t2, be_t2), (b_s, g_s, be_s),
        Wf1.T, Wf2.T, Wc1.T, bc1, gc1, bec1, Wc2.T, bc2, gc2, bec2,
        Wc3.T, bc3)
    return out_s.reshape(_B, _S, _DT), fused, cls


# speedup vs baseline: 1.0951x; 1.0951x over previous
import functools

import numpy as np
import jax
import jax.numpy as jnp
from jax import lax
from jax.experimental import pallas as pl
from jax.experimental.pallas import tpu as pltpu
from jax.experimental.pallas import tpu_sc as plsc

_B, _N, _S, _K = 4, 8192, 32, 12
_DS, _DT = 256, 1024
_G = _B * _S
_ROWS = _G * _K
_NW = 32
_RPW = _ROWS // _NW
_GPB = 16
_SQRT_C = float(np.sqrt(1.0 + 1e-5))


def _fps_knn_body(xt1_ref, xt2_ref, xs_ref, it1_ref, it2_ref, is_ref):
    x0 = xt1_ref[:, 0, :]
    x1 = xt1_ref[:, 1, :]
    x2 = xt1_ref[:, 2, :]
    iota = lax.broadcasted_iota(jnp.int32, (_B, _N), 1)
    dist = jnp.full((_B, _N), 1e10, dtype=jnp.float32)
    far = jnp.zeros((_B, 1), dtype=jnp.int32)
    qxs, qys, qzs = [], [], []
    for _ in range(_S):
        msk = iota == far
        c = jnp.sum(jnp.where(msk[:, None, :], xt1_ref[...], 0.0), axis=2)
        cx = c[:, 0:1]
        cy = c[:, 1:2]
        cz = c[:, 2:3]
        qxs.append(cx)
        qys.append(cy)
        qzs.append(cz)
        d = (x0 - cx) ** 2 + (x1 - cy) ** 2 + (x2 - cz) ** 2
        dist = jnp.minimum(dist, d)
        far = jnp.argmax(dist, axis=1).astype(jnp.int32)[:, None]
    qx = jnp.concatenate(qxs, axis=1)
    qy = jnp.concatenate(qys, axis=1)
    qz = jnp.concatenate(qzs, axis=1)
    sq = qx * qx + qy * qy + qz * qz
    qxf = qx.astype(jnp.bfloat16).astype(jnp.float32)
    qyf = qy.astype(jnp.bfloat16).astype(jnp.float32)
    qzf = qz.astype(jnp.bfloat16).astype(jnp.float32)
    iota3 = lax.broadcasted_iota(jnp.int32, (_B, _S, _N), 2)
    for db_ref, out_ref in ((xt1_ref, it1_ref), (xt2_ref, it2_ref),
                            (xs_ref, is_ref)):
        y0 = db_ref[:, 0, :]
        y1 = db_ref[:, 1, :]
        y2 = db_ref[:, 2, :]
        sx = y0 * y0 + y1 * y1 + y2 * y2
        y0f = y0.astype(jnp.bfloat16).astype(jnp.float32)
        y1f = y1.astype(jnp.bfloat16).astype(jnp.float32)
        y2f = y2.astype(jnp.bfloat16).astype(jnp.float32)
        p0 = qxf[:, :, None] * y0f[:, None, :]
        p1 = qyf[:, :, None] * y1f[:, None, :]
        p2 = qzf[:, :, None] * y2f[:, None, :]
        dot = (p0 + p1) + p2
        d = (sq[:, :, None] + sx[:, None, :]) - 2.0 * dot
        cols = []
        for _ in range(_K):
            ik = jnp.argmin(d, axis=2).astype(jnp.int32)[:, :, None]
            cols.append(ik)
            d = jnp.where(iota3 == ik, jnp.float32(jnp.inf), d)
        out_ref[...] = jnp.concatenate(cols, axis=2)


def _fps_knn(xt1, xt2, xs):
    out = jax.ShapeDtypeStruct((_B, _S, _K), jnp.int32)
    return pl.pallas_call(
        _fps_knn_body,
        out_shape=(out, out, out),
    )(xt1, xt2, xs)


def _sc_gather_body(t1_ref, t2_ref, ts_ref, idx_ref,
                    o1_ref, o2_ref, os_ref,
                    idx_v, r1_v, r2_v, rs_v,
                    sem1, sem2, sem3, semw1, semw2, semw3):
    wid = lax.axis_index("s") * 2 + lax.axis_index("c")
    base = wid * _RPW
    pltpu.sync_copy(idx_ref.at[pl.ds(wid * 3 * _RPW, 3 * _RPW)], idx_v)
    c1 = pltpu.async_copy(t1_ref.at[idx_v.at[pl.ds(0, _RPW)]], r1_v, sem1)
    c2 = pltpu.async_copy(t2_ref.at[idx_v.at[pl.ds(_RPW, _RPW)]], r2_v, sem2)
    c3 = pltpu.async_copy(ts_ref.at[idx_v.at[pl.ds(2 * _RPW, _RPW)]],
                          rs_v, sem3)
    c1.wait()
    w1 = pltpu.async_copy(r1_v, o1_ref.at[pl.ds(base, _RPW)], semw1)
    c2.wait()
    w2 = pltpu.async_copy(r2_v, o2_ref.at[pl.ds(base, _RPW)], semw2)
    c3.wait()
    w3 = pltpu.async_copy(rs_v, os_ref.at[pl.ds(base, _RPW)], semw3)
    w1.wait()
    w2.wait()
    w3.wait()


@functools.lru_cache(maxsize=1)
def _sc_gather_kernel():
    return pl.kernel(
        _sc_gather_body,
        out_type=(
            jax.ShapeDtypeStruct((_ROWS, _DT), jnp.float32),
            jax.ShapeDtypeStruct((_ROWS, _DT), jnp.float32),
            jax.ShapeDtypeStruct((_ROWS, _DS), jnp.float32),
        ),
        mesh=plsc.VectorSubcoreMesh(core_axis_name="c", subcore_axis_name="s"),
        scratch_types=[
            pltpu.VMEM((3 * _RPW,), jnp.int32),
            pltpu.VMEM((_RPW, _DT), jnp.float32),
            pltpu.VMEM((_RPW, _DT), jnp.float32),
            pltpu.VMEM((_RPW, _DS), jnp.float32),
            pltpu.SemaphoreType.DMA,
            pltpu.SemaphoreType.DMA,
            pltpu.SemaphoreType.DMA,
            pltpu.SemaphoreType.DMA,
            pltpu.SemaphoreType.DMA,
            pltpu.SemaphoreType.DMA,
        ],
    )


def _sc_gather(t1, t2, ts, idx):
    return _sc_gather_kernel()(t1, t2, ts, idx)


def _segmax(y):
    parts = [jnp.max(y[g * _K:(g + 1) * _K, :], axis=0, keepdims=True)
             for g in range(_GPB)]
    return jnp.concatenate(parts, axis=0)


def _bn_relu(y, b, g, be):
    y = y + b
    y = y / _SQRT_C * g + be
    return jnp.maximum(y, 0.0)


def _dense_body(g1_ref, g2_ref, gs_ref, wt1_ref, wt2_ref, wts_ref,
                b1_ref, g1s_ref, be1_ref, b2_ref, g2s_ref, be2_ref,
                bs_ref, gss_ref, bes_ref,
                wf1_ref, wf2_ref, wc1_ref, bc1_ref, gc1_ref, bec1_ref,
                wc2_ref, bc2_ref, gc2_ref, bec2_ref, wc3_ref, bc3_ref,
                os_ref, fused_ref, cls_ref, acc1_ref, acc2_ref):
    i = pl.program_id(0)
    y1 = jnp.dot(g1_ref[...].astype(jnp.bfloat16),
                 wt1_ref[...].astype(jnp.bfloat16),
                 preferred_element_type=jnp.float32)
    s1 = _segmax(_bn_relu(y1, b1_ref[...], g1s_ref[...], be1_ref[...]))
    acc1_ref[pl.ds(i * _GPB, _GPB), :] = s1
    y2 = jnp.dot(g2_ref[...].astype(jnp.bfloat16),
                 wt2_ref[...].astype(jnp.bfloat16),
                 preferred_element_type=jnp.float32)
    s2 = _segmax(_bn_relu(y2, b2_ref[...], g2s_ref[...], be2_ref[...]))
    acc2_ref[pl.ds(i * _GPB, _GPB), :] = s2
    ys = jnp.dot(gs_ref[...].astype(jnp.bfloat16),
                 wts_ref[...].astype(jnp.bfloat16),
                 preferred_element_type=jnp.float32)
    os_ref[...] = _segmax(_bn_relu(ys, bs_ref[...], gss_ref[...],
                                   bes_ref[...]))

    @pl.when(i == _G // _GPB - 1)
    def _():
        t1 = acc1_ref[...].reshape(_B, _S, _DT)
        t2 = acc2_ref[...].reshape(_B, _S, _DT)
        m1 = jnp.mean(t1, axis=1)
        m2 = jnp.mean(t2, axis=1)
        pm = jnp.concatenate([m1, m2], axis=1)
        y = jnp.dot(pm, wf1_ref[...], preferred_element_type=jnp.float32)
        y = jnp.maximum(y, 0.0)
        y = jnp.dot(y, wf2_ref[...], preferred_element_type=jnp.float32)
        mx = jnp.max(y, axis=1, keepdims=True)
        e = jnp.exp(y - mx)
        sm = e / jnp.sum(e, axis=1, keepdims=True)
        w0 = sm[:, 0:1][:, :, None]
        w1 = sm[:, 1:2][:, :, None]
        fused = w0 * t1 + w1 * t2
        fused_ref[...] = fused
        xm = jnp.max(fused, axis=1)
        h = jnp.dot(xm, wc1_ref[...], preferred_element_type=jnp.float32)
        h = h + bc1_ref[...]
        h = jnp.maximum(h / _SQRT_C * gc1_ref[...] + bec1_ref[...], 0.0)
        h = jnp.dot(h, wc2_ref[...], preferred_element_type=jnp.float32)
        h = h + bc2_ref[...]
        h = jnp.maximum(h / _SQRT_C * gc2_ref[...] + bec2_ref[...], 0.0)
        c = jnp.dot(h, wc3_ref[...], preferred_element_type=jnp.float32)
        cls_ref[...] = c + bc3_ref[...]


def _dense(g1, g2, gs, wt1, wt2, wts, bn1, bn2, bns,
           wf1t, wf2t, wc1t, bc1, gc1, bec1, wc2t, bc2, gc2, bec2,
           wc3t, bc3):
    row = pl.BlockSpec((1, _DT), lambda i: (0, 0))
    return pl.pallas_call(
        _dense_body,
        grid=(_G // _GPB,),
        in_specs=[
            pl.BlockSpec((_GPB * _K, _DT), lambda i: (i, 0)),
            pl.BlockSpec((_GPB * _K, _DT), lambda i: (i, 0)),
            pl.BlockSpec((_GPB * _K, _DS), lambda i: (i, 0)),
            pl.BlockSpec((_DT, _DT), lambda i: (0, 0)),
            pl.BlockSpec((_DT, _DT), lambda i: (0, 0)),
            pl.BlockSpec((_DS, _DT), lambda i: (0, 0)),
            row, row, row, row, row, row, row, row, row,
            pl.BlockSpec((2 * _DT, 128), lambda i: (0, 0)),
            pl.BlockSpec((128, 2), lambda i: (0, 0)),
            pl.BlockSpec((_DT, 512), lambda i: (0, 0)),
            pl.BlockSpec((1, 512), lambda i: (0, 0)),
            pl.BlockSpec((1, 512), lambda i: (0, 0)),
            pl.BlockSpec((1, 512), lambda i: (0, 0)),
            pl.BlockSpec((512, 256), lambda i: (0, 0)),
            pl.BlockSpec((1, 256), lambda i: (0, 0)),
            pl.BlockSpec((1, 256), lambda i: (0, 0)),
            pl.BlockSpec((1, 256), lambda i: (0, 0)),
            pl.BlockSpec((256, 15), lambda i: (0, 0)),
            pl.BlockSpec((1, 15), lambda i: (0, 0)),
        ],
        out_specs=(
            pl.BlockSpec((_GPB, _DT), lambda i: (i, 0)),
            pl.BlockSpec((_B, _S, _DT), lambda i: (0, 0, 0)),
            pl.BlockSpec((_B, 15), lambda i: (0, 0)),
        ),
        out_shape=(
            jax.ShapeDtypeStruct((_G, _DT), jnp.float32),
            jax.ShapeDtypeStruct((_B, _S, _DT), jnp.float32),
            jax.ShapeDtypeStruct((_B, 15), jnp.float32),
        ),
        scratch_shapes=[
            pltpu.VMEM((_G, _DT), jnp.float32),
            pltpu.VMEM((_G, _DT), jnp.float32),
        ],
    )(g1, g2, gs, wt1, wt2, wts,
      bn1[0].reshape(1, -1), bn1[1].reshape(1, -1), bn1[2].reshape(1, -1),
      bn2[0].reshape(1, -1), bn2[1].reshape(1, -1), bn2[2].reshape(1, -1),
      bns[0].reshape(1, -1), bns[1].reshape(1, -1), bns[2].reshape(1, -1),
      wf1t, wf2t, wc1t, bc1.reshape(1, -1), gc1.reshape(1, -1),
      bec1.reshape(1, -1), wc2t, bc2.reshape(1, -1), gc2.reshape(1, -1),
      bec2.reshape(1, -1), wc3t, bc3.reshape(1, -1))


def kernel(feature_s, xyz_s, feature_t1, xyz_t1, feature_t2, xyz_t2,
           W_s, b_s, g_s, be_s, W_t1, b_t1, g_t1, be_t1,
           W_t2, b_t2, g_t2, be_t2, Wf1, Wf2, Wc1, bc1, gc1, bec1,
           Wc2, bc2, gc2, bec2, Wc3, bc3):
    xt1 = jnp.transpose(xyz_t1, (0, 2, 1))
    xt2 = jnp.transpose(xyz_t2, (0, 2, 1))
    xs = jnp.transpose(xyz_s, (0, 2, 1))
    idx_t1, idx_t2, idx_s = _fps_knn(xt1, xt2, xs)
    off = (jnp.arange(_B, dtype=jnp.int32) * _N)[:, None, None]
    f1 = (idx_t1 + off).reshape(_NW, _RPW)
    f2 = (idx_t2 + off).reshape(_NW, _RPW)
    fs = (idx_s + off).reshape(_NW, _RPW)
    idx = jnp.stack([f1, f2, fs], axis=1).reshape(-1)
    g1, g2, gs = _sc_gather(
        feature_t1.reshape(_B * _N, _DT),
        feature_t2.reshape(_B * _N, _DT),
        feature_s.reshape(_B * _N, _DS),
        idx)
    out_s, fused, cls = _dense(
        g1, g2, gs, W_t1.T, W_t2.T, W_s.T,
        (b_t1, g_t1, be_t1), (b_t2, g_t2, be_t2), (b_s, g_s, be_s),
        Wf1.T, Wf2.T, Wc1.T, bc1, gc1, bec1, Wc2.T, bc2, gc2, bec2,
        Wc3.T, bc3)
    return out_s.reshape(_B, _S, _DT), fused, cls
